# baseline (speedup 1.0000x reference)
import jax
import jax.numpy as jnp
from jax.experimental import pallas as pl
from jax.experimental.pallas import tpu as pltpu

NUM_OPS = 8
TN = 512


def _body(x_ref, mask_ref, alphas_ref, w_ref, b_ref, o_ref):
    i = pl.program_id(1)

    a = alphas_ref[...]
    a = a - jnp.max(a)
    e = jnp.exp(a)
    p = e / jnp.sum(e)
    lane = jax.lax.broadcasted_iota(jnp.int32, (1, NUM_OPS), 1)
    p_i = jnp.sum(jnp.where(lane == i, p, 0.0))

    w16 = (w_ref[0] * p_i).astype(jnp.bfloat16)
    acc = jnp.dot(x_ref[...], w16, preferred_element_type=jnp.float32)
    maskcol = mask_ref[...].astype(jnp.float32)
    val = jnp.maximum(acc * maskcol + p_i * b_ref[0], 0.0)

    @pl.when(i == 0)
    def _init():
        o_ref[...] = val

    @pl.when(i > 0)
    def _acc():
        o_ref[...] += val


@jax.jit
def kernel(x, mask, alphas, W, b):
    n_tok, d_model = x.shape
    num_ops = W.shape[0]
    x16 = x.astype(jnp.bfloat16)
    mask2d = mask.reshape(n_tok, 1)
    alphas2d = alphas.reshape(1, num_ops)
    b3d = b.reshape(num_ops, 1, d_model)

    grid = (d_model // TN, num_ops)
    out = pl.pallas_call(
        _body,
        grid=grid,
        in_specs=[
            pl.BlockSpec((n_tok, d_model), lambda n, i: (0, 0)),
            pl.BlockSpec((n_tok, 1), lambda n, i: (0, 0)),
            pl.BlockSpec((1, num_ops), lambda n, i: (0, 0)),
            pl.BlockSpec((1, d_model, TN), lambda n, i: (i, 0, n)),
            pl.BlockSpec((1, 1, TN), lambda n, i: (i, 0, n)),
        ],
        out_specs=pl.BlockSpec((n_tok, TN), lambda n, i: (0, n)),
        out_shape=jax.ShapeDtypeStruct((n_tok, d_model), jnp.float32),
        compiler_params=pltpu.CompilerParams(
            dimension_semantics=("parallel", "arbitrary"),
        ),
    )(x16, mask2d, alphas2d, W, b3d)
    return out
